# Initial kernel scaffold; baseline (speedup 1.0000x reference)
import functools

import jax
import jax.numpy as jnp
from jax import lax
from jax.experimental import pallas as pl
from jax.experimental.pallas import tpu as pltpu
from jax.experimental.pallas import tpu_sc as plsc

N = 10000
E = 320000
NT = 16
RPT = N // NT
K = 100
C = (E // NT) // K
C2 = (E // 2 // NT) // K
BM = 2000
NB = N // BM


def _deg_kernel(idx, ones, zeros):
    mesh = plsc.VectorSubcoreMesh(core_axis_name="c", subcore_axis_name="s")

    @functools.partial(
        pl.kernel,
        out_type=jax.ShapeDtypeStruct((2, N, 16), jnp.float32),
        mesh=mesh,
        scratch_types=[
            pltpu.VMEM((C2, K), jnp.int32),
            pltpu.VMEM((K, 16), jnp.float32),
            pltpu.VMEM_SHARED((N, 16), jnp.float32),
        ],
    )
    def k(idx_hbm, ones_hbm, z_hbm, out_hbm, idx_v, ones_v, acc_sm):
        c = lax.axis_index("c")
        s = lax.axis_index("s")
        pltpu.sync_copy(idx_hbm.at[c, s], idx_v)
        pltpu.sync_copy(ones_hbm, ones_v)
        pltpu.sync_copy(z_hbm.at[pl.ds(s * RPT, RPT)],
                        acc_sm.at[pl.ds(s * RPT, RPT)])
        plsc.subcore_barrier()

        def body(j, carry):
            pltpu.sync_copy(ones_v, acc_sm.at[idx_v.at[j]], add=True)
            return carry

        lax.fori_loop(0, C2, body, 0)
        plsc.subcore_barrier()
        pltpu.sync_copy(acc_sm.at[pl.ds(s * RPT, RPT)],
                        out_hbm.at[c, pl.ds(s * RPT, RPT)])

    return k(idx, ones, zeros)


def _edge_scatter(D, hcat, isrc2, idst, zeros):
    mesh = plsc.VectorSubcoreMesh(core_axis_name="c", subcore_axis_name="s")

    @functools.partial(
        pl.kernel,
        out_type=jax.ShapeDtypeStruct((2, N, D), jnp.float32),
        mesh=mesh,
        scratch_types=[
            pltpu.VMEM((C, K), jnp.int32),
            pltpu.VMEM((C, K), jnp.int32),
            pltpu.VMEM((K, D), jnp.float32),
            pltpu.SemaphoreType.DMA,
            pltpu.VMEM_SHARED((N, D), jnp.float32),
        ],
    )
    def k(h_hbm, isrc_hbm, idst_hbm, z_hbm, out_hbm,
          isrc_v, idst_v, rows_v, sem, acc_sm):
        c = lax.axis_index("c")
        s = lax.axis_index("s")
        pltpu.sync_copy(isrc_hbm.at[c, s], isrc_v)
        pltpu.sync_copy(idst_hbm.at[s], idst_v)
        pltpu.sync_copy(z_hbm.at[pl.ds(s * RPT, RPT)],
                        acc_sm.at[pl.ds(s * RPT, RPT)])
        plsc.subcore_barrier()

        def body(j, carry):
            pltpu.async_copy(h_hbm.at[isrc_v.at[j]], rows_v, sem).wait()
            pltpu.sync_copy(rows_v, acc_sm.at[idst_v.at[j]], add=True)
            return carry

        lax.fori_loop(0, C, body, 0)
        plsc.subcore_barrier()
        pltpu.sync_copy(acc_sm.at[pl.ds(s * RPT, RPT)],
                        out_hbm.at[c, pl.ds(s * RPT, RPT)])

    return k(hcat, isrc2, idst, zeros)


def _t1_body(x_ref, w_ref, deg_ref, h_ref, dinv_ref):
    deg = deg_ref[0] + deg_ref[1]
    degt = jnp.sum(deg, axis=1, keepdims=True) + 1.0
    dinv = lax.rsqrt(degt)
    h = jnp.dot(x_ref[...], w_ref[...], preferred_element_type=jnp.float32)
    h_ref[...] = h * dinv
    dinv_ref[...] = dinv


def _t2_body(acc_ref, h1_ref, dinv_ref, b1_ref, w2_ref, out_ref):
    kk = pl.program_id(1)
    dinv = dinv_ref[...]
    z = (acc_ref[...] + h1_ref[...]) * dinv + b1_ref[...]
    a = jnp.maximum(z, 0.0)
    part = jnp.dot(a, w2_ref[...], preferred_element_type=jnp.float32) * dinv

    @pl.when(kk == 0)
    def _():
        out_ref[...] = part

    @pl.when(kk == 1)
    def _():
        out_ref[...] += part


def _t3_body(acc_ref, h2_ref, dinv_ref, b2_ref, out_ref):
    r = (acc_ref[...] + h2_ref[...]) * dinv_ref[...] + b2_ref[...]
    out_ref[...] = r[None]


def kernel(x, edge_index, W1, b1, W2, b2):
    src = edge_index[0].astype(jnp.int32)
    dst = edge_index[1].astype(jnp.int32)
    isrc2 = jnp.stack([src, src + N]).reshape(2, NT, C, K)
    idst = dst.reshape(NT, C, K)
    idst_deg = dst.reshape(2, NT, C2, K)
    ones16 = jnp.ones((K, 16), jnp.float32)
    z16 = jnp.zeros((N, 16), jnp.float32)
    z128 = jnp.zeros((N, 128), jnp.float32)
    z64 = jnp.zeros((N, 64), jnp.float32)
    b1r = b1.reshape(2, 128)
    b2r = b2.reshape(2, 64)

    deg = _deg_kernel(idst_deg, ones16, z16)

    h1cat, dinv = pl.pallas_call(
        _t1_body,
        grid=(NB, 2),
        in_specs=[
            pl.BlockSpec((BM, 128), lambda i, j: (i, 0)),
            pl.BlockSpec((128, 128), lambda i, j: (0, j)),
            pl.BlockSpec((2, BM, 16), lambda i, j: (0, i, 0)),
        ],
        out_specs=[
            pl.BlockSpec((BM, 128), lambda i, j: (j * NB + i, 0)),
            pl.BlockSpec((BM, 1), lambda i, j: (i, 0)),
        ],
        out_shape=[
            jax.ShapeDtypeStruct((2 * N, 128), jnp.float32),
            jax.ShapeDtypeStruct((N, 1), jnp.float32),
        ],
    )(x, W1, deg)

    acc1 = _edge_scatter(128, h1cat, isrc2, idst, z128)
    acc1cat = acc1.reshape(2 * N, 128)

    h2 = pl.pallas_call(
        _t2_body,
        grid=(NB, 2),
        in_specs=[
            pl.BlockSpec((BM, 128), lambda i, k: (k * NB + i, 0)),
            pl.BlockSpec((BM, 128), lambda i, k: (k * NB + i, 0)),
            pl.BlockSpec((BM, 1), lambda i, k: (i, 0)),
            pl.BlockSpec((1, 128), lambda i, k: (k, 0)),
            pl.BlockSpec((128, 128), lambda i, k: (k, 0)),
        ],
        out_specs=pl.BlockSpec((BM, 128), lambda i, k: (i, 0)),
        out_shape=jax.ShapeDtypeStruct((N, 128), jnp.float32),
    )(acc1cat, h1cat, dinv, b1r, W2)

    h2cat = jnp.concatenate([h2[:, :64], h2[:, 64:]], axis=0)

    acc2 = _edge_scatter(64, h2cat, isrc2, idst, z64)
    acc2cat = acc2.reshape(2 * N, 64)

    o3 = pl.pallas_call(
        _t3_body,
        grid=(NB, 2),
        in_specs=[
            pl.BlockSpec((BM, 64), lambda i, j: (j * NB + i, 0)),
            pl.BlockSpec((BM, 64), lambda i, j: (j * NB + i, 0)),
            pl.BlockSpec((BM, 1), lambda i, j: (i, 0)),
            pl.BlockSpec((1, 64), lambda i, j: (j, 0)),
        ],
        out_specs=pl.BlockSpec((1, BM, 64), lambda i, j: (j, i, 0)),
        out_shape=jax.ShapeDtypeStruct((2, N, 64), jnp.float32),
    )(acc2cat, h2cat, dinv, b2r)

    return jnp.concatenate([o3[0], o3[1]], axis=1)


# speedup vs baseline: 11.9969x; 11.9969x over previous
import functools

import jax
import jax.numpy as jnp
from jax import lax
from jax.experimental import pallas as pl
from jax.experimental.pallas import tpu as pltpu
from jax.experimental.pallas import tpu_sc as plsc

N = 10000
E = 320000
NT = 16
RPT = N // NT
K = 100
C = (E // NT) // K
C2 = (E // 2 // NT) // K
W = 64
BM = 2000
NB = N // BM


def _deg_kernel(idx, ones, zeros):
    mesh = plsc.VectorSubcoreMesh(core_axis_name="c", subcore_axis_name="s")

    @functools.partial(
        pl.kernel,
        out_type=jax.ShapeDtypeStruct((2, NT, RPT, 16), jnp.float32),
        mesh=mesh,
        compiler_params=pltpu.CompilerParams(use_tc_tiling_on_sc=False),
        scratch_types=[
            pltpu.VMEM((C2, K), jnp.int32),
            pltpu.VMEM((K, 16), jnp.float32),
            pltpu.VMEM_SHARED((N, 16), jnp.float32),
        ],
    )
    def k(idx_hbm, ones_hbm, z_hbm, out_hbm, idx_v, ones_v, acc_sm):
        c = lax.axis_index("c")
        s = lax.axis_index("s")
        pltpu.sync_copy(idx_hbm.at[c, s], idx_v)
        pltpu.sync_copy(ones_hbm, ones_v)
        pltpu.sync_copy(z_hbm.at[s], acc_sm.at[pl.ds(s * RPT, RPT)])
        plsc.subcore_barrier()

        def body(j, carry):
            pltpu.sync_copy(ones_v, acc_sm.at[idx_v.at[j]], add=True)
            return carry

        lax.fori_loop(0, C2, body, 0)
        plsc.subcore_barrier()
        pltpu.sync_copy(acc_sm.at[pl.ds(s * RPT, RPT)], out_hbm.at[c, s])

    return k(idx, ones, zeros)


def _edge_scatter_jnp(NP, hq, isrcq, idst, zeros):
    src = isrcq.reshape(2, NP, E)
    d = idst.reshape(E)
    out = jnp.zeros((2, NP, N, W), jnp.float32)
    out = out.at[:, :, d, :].add(hq[src], mode="drop")
    return out.reshape(2, NP, NT, RPT, W)


def _edge_scatter(NP, hq, isrcq, idst, zeros):
    mesh = plsc.VectorSubcoreMesh(core_axis_name="c", subcore_axis_name="s")

    @functools.partial(
        pl.kernel,
        out_type=jax.ShapeDtypeStruct((2, NP, NT, RPT, W), jnp.float32),
        mesh=mesh,
        compiler_params=pltpu.CompilerParams(use_tc_tiling_on_sc=False),
        scratch_types=[
            pltpu.VMEM((C, K), jnp.int32),
            pltpu.VMEM((C, K), jnp.int32),
            pltpu.VMEM((K, W), jnp.float32),
            pltpu.SemaphoreType.DMA,
            pltpu.VMEM_SHARED((N, W), jnp.float32),
        ],
    )
    def k(h_hbm, isrc_hbm, idst_hbm, z_hbm, out_hbm,
          isrc_v, idst_v, rows_v, sem, acc_sm):
        c = lax.axis_index("c")
        s = lax.axis_index("s")
        pltpu.sync_copy(idst_hbm.at[s], idst_v)
        for q in range(NP):
            pltpu.sync_copy(isrc_hbm.at[c, q, s], isrc_v)
            pltpu.sync_copy(z_hbm.at[s], acc_sm.at[pl.ds(s * RPT, RPT)])
            plsc.subcore_barrier()

            def body(j, carry):
                pltpu.async_copy(h_hbm.at[isrc_v.at[j]], rows_v, sem).wait()
                pltpu.sync_copy(rows_v, acc_sm.at[idst_v.at[j]], add=True)
                return carry

            lax.fori_loop(0, C, body, 0)
            plsc.subcore_barrier()
            pltpu.sync_copy(acc_sm.at[pl.ds(s * RPT, RPT)], out_hbm.at[c, q, s])

    return k(hq, isrcq, idst, zeros)


def _t1_body(x_ref, w_ref, deg_ref, h_ref, dinv_ref):
    deg = deg_ref[0] + deg_ref[1]
    degt = deg[:, 0:1] + 1.0
    dinv = lax.rsqrt(degt)
    h = lax.dot_general(x_ref[...], w_ref[...], (((1,), (1,)), ((), ())),
                        preferred_element_type=jnp.float32)
    h_ref[...] = h * dinv
    dinv_ref[...] = dinv


def _t2_body(a0_ref, a1_ref, h0_ref, h1_ref, dinv_ref, b1_ref, w2_ref, out_ref):
    kk = pl.program_id(1)
    dinv = dinv_ref[...]
    b = b1_ref[...]
    bk = jnp.where(kk == 0, b[0:1], b[1:2])
    z = jnp.concatenate(
        [a0_ref[...] + h0_ref[...], a1_ref[...] + h1_ref[...]], axis=1)
    z = z * dinv + bk
    a = jnp.maximum(z, 0.0)
    part = jnp.dot(a, w2_ref[...], preferred_element_type=jnp.float32) * dinv

    @pl.when(kk == 0)
    def _():
        out_ref[...] = part

    @pl.when(kk == 1)
    def _():
        out_ref[...] += part


def _t3_body(acc_ref, h2_ref, dinv_ref, b2_ref, out_ref):
    jj = pl.program_id(1)
    b = b2_ref[...]
    bj = jnp.where(jj == 0, b[0:1], b[1:2])
    r = (acc_ref[...] + h2_ref[...]) * dinv_ref[...] + bj
    out_ref[...] = r[None]


def kernel(x, edge_index, W1, b1, W2, b2):
    src = edge_index[0].astype(jnp.int32)
    dst = edge_index[1].astype(jnp.int32)
    offs4 = jnp.arange(4, dtype=jnp.int32).reshape(2, 2) * N
    isrc4 = (src[None, None] + offs4[:, :, None]).reshape(2, 2, NT, C, K)
    offs2 = jnp.arange(2, dtype=jnp.int32).reshape(2, 1) * N
    isrc2 = (src[None, None] + offs2[:, :, None]).reshape(2, 1, NT, C, K)
    idst = dst.reshape(NT, C, K)
    idst_deg = dst.reshape(2, NT, C2, K)
    ones16 = jnp.ones((K, 16), jnp.float32)
    z16 = jnp.zeros((NT, RPT, 16), jnp.float32)
    zW = jnp.zeros((NT, RPT, W), jnp.float32)
    b1r = b1.reshape(2, 128)
    b2r = b2.reshape(2, 64)
    W1T = W1.T.reshape(4 * 64, 128)

    deg = _deg_kernel(idst_deg, ones16, z16).reshape(2, N, 16)

    h1q, dinv = pl.pallas_call(
        _t1_body,
        grid=(NB, 4),
        in_specs=[
            pl.BlockSpec((BM, 128), lambda i, j: (i, 0)),
            pl.BlockSpec((64, 128), lambda i, j: (j, 0)),
            pl.BlockSpec((2, BM, 16), lambda i, j: (0, i, 0)),
        ],
        out_specs=[
            pl.BlockSpec((BM, 64), lambda i, j: (j * NB + i, 0)),
            pl.BlockSpec((BM, 1), lambda i, j: (i, 0)),
        ],
        out_shape=[
            jax.ShapeDtypeStruct((4 * N, 64), jnp.float32),
            jax.ShapeDtypeStruct((N, 1), jnp.float32),
        ],
    )(x, W1T, deg)

    acc1q = _edge_scatter(2, h1q, isrc4, idst, zW).reshape(4 * N, 64)

    h2 = pl.pallas_call(
        _t2_body,
        grid=(NB, 2),
        in_specs=[
            pl.BlockSpec((BM, 64), lambda i, k: (2 * k * NB + i, 0)),
            pl.BlockSpec((BM, 64), lambda i, k: ((2 * k + 1) * NB + i, 0)),
            pl.BlockSpec((BM, 64), lambda i, k: (2 * k * NB + i, 0)),
            pl.BlockSpec((BM, 64), lambda i, k: ((2 * k + 1) * NB + i, 0)),
            pl.BlockSpec((BM, 1), lambda i, k: (i, 0)),
            pl.BlockSpec((2, 128), lambda i, k: (0, 0)),
            pl.BlockSpec((128, 128), lambda i, k: (k, 0)),
        ],
        out_specs=pl.BlockSpec((BM, 128), lambda i, k: (i, 0)),
        out_shape=jax.ShapeDtypeStruct((N, 128), jnp.float32),
    )(acc1q, acc1q, h1q, h1q, dinv, b1r, W2)

    h2cat = jnp.concatenate([h2[:, :64], h2[:, 64:]], axis=0)

    acc2cat = _edge_scatter(1, h2cat, isrc2, idst, zW).reshape(2 * N, 64)

    o3 = pl.pallas_call(
        _t3_body,
        grid=(NB, 2),
        in_specs=[
            pl.BlockSpec((BM, 64), lambda i, j: (j * NB + i, 0)),
            pl.BlockSpec((BM, 64), lambda i, j: (j * NB + i, 0)),
            pl.BlockSpec((BM, 1), lambda i, j: (i, 0)),
            pl.BlockSpec((2, 64), lambda i, j: (0, 0)),
        ],
        out_specs=pl.BlockSpec((1, BM, 64), lambda i, j: (j, i, 0)),
        out_shape=jax.ShapeDtypeStruct((2, N, 64), jnp.float32),
    )(acc2cat, h2cat, dinv, b2r)

    return jnp.concatenate([o3[0], o3[1]], axis=1)


# speedup vs baseline: 18.7839x; 1.5657x over previous
import functools

import jax
import jax.numpy as jnp
from jax import lax
from jax.experimental import pallas as pl
from jax.experimental.pallas import tpu as pltpu
from jax.experimental.pallas import tpu_sc as plsc

N = 10000
E = 320000
NT = 16
RPT = N // NT
K = 100
C = (E // NT) // K
C2 = (E // 2 // NT) // K
W = 64
BM = 2000
NB = N // BM


def _deg_kernel(idx, ones, zeros):
    mesh = plsc.VectorSubcoreMesh(core_axis_name="c", subcore_axis_name="s")

    @functools.partial(
        pl.kernel,
        out_type=jax.ShapeDtypeStruct((2, NT, RPT, 16), jnp.float32),
        mesh=mesh,
        compiler_params=pltpu.CompilerParams(use_tc_tiling_on_sc=False),
        scratch_types=[
            pltpu.VMEM((C2, K), jnp.int32),
            pltpu.VMEM((K, 16), jnp.float32),
            pltpu.VMEM_SHARED((N, 16), jnp.float32),
        ],
    )
    def k(idx_hbm, ones_hbm, z_hbm, out_hbm, idx_v, ones_v, acc_sm):
        c = lax.axis_index("c")
        s = lax.axis_index("s")
        pltpu.sync_copy(idx_hbm.at[c, s], idx_v)
        pltpu.sync_copy(ones_hbm, ones_v)
        pltpu.sync_copy(z_hbm.at[s], acc_sm.at[pl.ds(s * RPT, RPT)])
        plsc.subcore_barrier()

        def body(j, carry):
            pltpu.sync_copy(ones_v, acc_sm.at[idx_v.at[j]], add=True)
            return carry

        lax.fori_loop(0, C2, body, 0)
        plsc.subcore_barrier()
        pltpu.sync_copy(acc_sm.at[pl.ds(s * RPT, RPT)], out_hbm.at[c, s])

    return k(idx, ones, zeros)


def _edge_scatter_jnp(NP, hq, isrcq, idst, zeros):
    src = isrcq.reshape(2, NP, E)
    d = idst.reshape(E)
    out = jnp.zeros((2, NP, N, W), jnp.float32)
    out = out.at[:, :, d, :].add(hq[src], mode="drop")
    return out.reshape(2, NP, NT, RPT, W)


def _edge_scatter(NP, hq, isrcq, idst, zeros):
    mesh = plsc.VectorSubcoreMesh(core_axis_name="c", subcore_axis_name="s")

    @functools.partial(
        pl.kernel,
        out_type=jax.ShapeDtypeStruct((2, NP, NT, RPT, W), jnp.float32),
        mesh=mesh,
        compiler_params=pltpu.CompilerParams(use_tc_tiling_on_sc=False),
        scratch_types=[
            pltpu.VMEM((C, K), jnp.int32),
            pltpu.VMEM((C, K), jnp.int32),
            pltpu.VMEM((K, W), jnp.float32),
            pltpu.VMEM((K, W), jnp.float32),
            pltpu.SemaphoreType.DMA,
            pltpu.SemaphoreType.DMA,
            pltpu.VMEM_SHARED((N, W), jnp.float32),
        ],
    )
    def k(h_hbm, isrc_hbm, idst_hbm, z_hbm, out_hbm,
          isrc_v, idst_v, ra_v, rb_v, sem_a, sem_b, acc_sm):
        c = lax.axis_index("c")
        s = lax.axis_index("s")
        pltpu.sync_copy(idst_hbm.at[s], idst_v)
        for q in range(NP):
            pltpu.sync_copy(isrc_hbm.at[c, q, s], isrc_v)
            pltpu.sync_copy(z_hbm.at[s], acc_sm.at[pl.ds(s * RPT, RPT)])
            pltpu.async_copy(h_hbm.at[isrc_v.at[0]], ra_v, sem_a)
            plsc.subcore_barrier()

            def body(jj, carry):
                j = 2 * jj
                pltpu.async_copy(h_hbm.at[isrc_v.at[j + 1]], rb_v, sem_b)
                pltpu.make_async_copy(h_hbm.at[isrc_v.at[j]], ra_v, sem_a).wait()
                pltpu.sync_copy(ra_v, acc_sm.at[idst_v.at[j]], add=True)

                @pl.when(jj < C // 2 - 1)
                def _():
                    pltpu.async_copy(h_hbm.at[isrc_v.at[j + 2]], ra_v, sem_a)

                pltpu.make_async_copy(h_hbm.at[isrc_v.at[j + 1]], rb_v, sem_b).wait()
                pltpu.sync_copy(rb_v, acc_sm.at[idst_v.at[j + 1]], add=True)
                return carry

            lax.fori_loop(0, C // 2, body, 0)
            plsc.subcore_barrier()
            pltpu.sync_copy(acc_sm.at[pl.ds(s * RPT, RPT)], out_hbm.at[c, q, s])

    return k(hq, isrcq, idst, zeros)


def _t1_body(x_ref, w_ref, deg_ref, h_ref, dinv_ref):
    deg = deg_ref[0] + deg_ref[1]
    degt = deg[:, 0:1] + 1.0
    dinv = lax.rsqrt(degt)
    h = lax.dot_general(x_ref[...], w_ref[...], (((1,), (1,)), ((), ())),
                        preferred_element_type=jnp.float32)
    h_ref[...] = h * dinv
    dinv_ref[...] = dinv


def _t2_body(a0_ref, a1_ref, h0_ref, h1_ref, dinv_ref, b1_ref, w2_ref, out_ref):
    kk = pl.program_id(1)
    dinv = dinv_ref[...]
    b = b1_ref[...]
    bk = jnp.where(kk == 0, b[0:1], b[1:2])
    z = jnp.concatenate(
        [a0_ref[...] + h0_ref[...], a1_ref[...] + h1_ref[...]], axis=1)
    z = z * dinv + bk
    a = jnp.maximum(z, 0.0)
    part = jnp.dot(a, w2_ref[...], preferred_element_type=jnp.float32) * dinv

    @pl.when(kk == 0)
    def _():
        out_ref[...] = part

    @pl.when(kk == 1)
    def _():
        out_ref[...] += part


def _t3_body(acc_ref, h2_ref, dinv_ref, b2_ref, out_ref):
    jj = pl.program_id(1)
    b = b2_ref[...]
    bj = jnp.where(jj == 0, b[0:1], b[1:2])
    r = (acc_ref[...] + h2_ref[...]) * dinv_ref[...] + bj
    out_ref[...] = r[None]


def kernel(x, edge_index, W1, b1, W2, b2):
    src = edge_index[0].astype(jnp.int32)
    dst = edge_index[1].astype(jnp.int32)
    offs4 = jnp.arange(4, dtype=jnp.int32).reshape(2, 2) * N
    isrc4 = (src[None, None] + offs4[:, :, None]).reshape(2, 2, NT, C, K)
    offs2 = jnp.arange(2, dtype=jnp.int32).reshape(2, 1) * N
    isrc2 = (src[None, None] + offs2[:, :, None]).reshape(2, 1, NT, C, K)
    idst = dst.reshape(NT, C, K)
    idst_deg = dst.reshape(2, NT, C2, K)
    ones16 = jnp.ones((K, 16), jnp.float32)
    z16 = jnp.zeros((NT, RPT, 16), jnp.float32)
    zW = jnp.zeros((NT, RPT, W), jnp.float32)
    b1r = b1.reshape(2, 128)
    b2r = b2.reshape(2, 64)
    W1T = W1.T.reshape(4 * 64, 128)

    deg = _deg_kernel(idst_deg, ones16, z16).reshape(2, N, 16)

    h1q, dinv = pl.pallas_call(
        _t1_body,
        grid=(NB, 4),
        in_specs=[
            pl.BlockSpec((BM, 128), lambda i, j: (i, 0)),
            pl.BlockSpec((64, 128), lambda i, j: (j, 0)),
            pl.BlockSpec((2, BM, 16), lambda i, j: (0, i, 0)),
        ],
        out_specs=[
            pl.BlockSpec((BM, 64), lambda i, j: (j * NB + i, 0)),
            pl.BlockSpec((BM, 1), lambda i, j: (i, 0)),
        ],
        out_shape=[
            jax.ShapeDtypeStruct((4 * N, 64), jnp.float32),
            jax.ShapeDtypeStruct((N, 1), jnp.float32),
        ],
    )(x, W1T, deg)

    acc1q = _edge_scatter(2, h1q, isrc4, idst, zW).reshape(4 * N, 64)

    h2 = pl.pallas_call(
        _t2_body,
        grid=(NB, 2),
        in_specs=[
            pl.BlockSpec((BM, 64), lambda i, k: (2 * k * NB + i, 0)),
            pl.BlockSpec((BM, 64), lambda i, k: ((2 * k + 1) * NB + i, 0)),
            pl.BlockSpec((BM, 64), lambda i, k: (2 * k * NB + i, 0)),
            pl.BlockSpec((BM, 64), lambda i, k: ((2 * k + 1) * NB + i, 0)),
            pl.BlockSpec((BM, 1), lambda i, k: (i, 0)),
            pl.BlockSpec((2, 128), lambda i, k: (0, 0)),
            pl.BlockSpec((128, 128), lambda i, k: (k, 0)),
        ],
        out_specs=pl.BlockSpec((BM, 128), lambda i, k: (i, 0)),
        out_shape=jax.ShapeDtypeStruct((N, 128), jnp.float32),
    )(acc1q, acc1q, h1q, h1q, dinv, b1r, W2)

    h2cat = jnp.concatenate([h2[:, :64], h2[:, 64:]], axis=0)

    acc2cat = _edge_scatter(1, h2cat, isrc2, idst, zW).reshape(2 * N, 64)

    o3 = pl.pallas_call(
        _t3_body,
        grid=(NB, 2),
        in_specs=[
            pl.BlockSpec((BM, 64), lambda i, j: (j * NB + i, 0)),
            pl.BlockSpec((BM, 64), lambda i, j: (j * NB + i, 0)),
            pl.BlockSpec((BM, 1), lambda i, j: (i, 0)),
            pl.BlockSpec((2, 64), lambda i, j: (0, 0)),
        ],
        out_specs=pl.BlockSpec((1, BM, 64), lambda i, j: (j, i, 0)),
        out_shape=jax.ShapeDtypeStruct((2, N, 64), jnp.float32),
    )(acc2cat, h2cat, dinv, b2r)

    return jnp.concatenate([o3[0], o3[1]], axis=1)


# speedup vs baseline: 30.9359x; 1.6469x over previous
import functools

import jax
import jax.numpy as jnp
from jax import lax
from jax.experimental import pallas as pl
from jax.experimental.pallas import tpu as pltpu
from jax.experimental.pallas import tpu_sc as plsc

N = 10000
E = 320000
NT = 16
RPT = N // NT
K = 125
C = (E // NT) // K
C2 = (E // 2 // NT) // K
W = 64
BM = 2000
NB = N // BM


def _deg_kernel(idx, ones, zeros):
    mesh = plsc.VectorSubcoreMesh(core_axis_name="c", subcore_axis_name="s")

    @functools.partial(
        pl.kernel,
        out_type=jax.ShapeDtypeStruct((2, NT, RPT, 16), jnp.float32),
        mesh=mesh,
        compiler_params=pltpu.CompilerParams(use_tc_tiling_on_sc=False),
        scratch_types=[
            pltpu.VMEM((C2, K), jnp.int32),
            pltpu.VMEM((K, 16), jnp.float32),
            pltpu.SemaphoreType.DMA,
            pltpu.VMEM_SHARED((N, 16), jnp.float32),
        ],
    )
    def k(idx_hbm, ones_hbm, z_hbm, out_hbm, idx_v, ones_v, sem, acc_sm):
        c = lax.axis_index("c")
        s = lax.axis_index("s")
        pltpu.sync_copy(idx_hbm.at[c, s], idx_v)
        pltpu.sync_copy(ones_hbm, ones_v)
        pltpu.sync_copy(z_hbm.at[s], acc_sm.at[pl.ds(s * RPT, RPT)])
        plsc.subcore_barrier()

        def body(j, carry):
            pltpu.async_copy(ones_v, acc_sm.at[idx_v.at[j]], sem, add=True)
            return carry

        lax.fori_loop(0, C2, body, 0)

        def drain(j, carry):
            pltpu.make_async_copy(ones_v, acc_sm.at[idx_v.at[0]], sem).wait()
            return carry

        lax.fori_loop(0, C2, drain, 0)
        plsc.subcore_barrier()
        pltpu.sync_copy(acc_sm.at[pl.ds(s * RPT, RPT)], out_hbm.at[c, s])

    return k(idx, ones, zeros)


def _edge_scatter(NP, hq, isrcq, idst, zeros):
    mesh = plsc.VectorSubcoreMesh(core_axis_name="c", subcore_axis_name="s")

    @functools.partial(
        pl.kernel,
        out_type=jax.ShapeDtypeStruct((2, NP, NT, RPT, W), jnp.float32),
        mesh=mesh,
        compiler_params=pltpu.CompilerParams(use_tc_tiling_on_sc=False),
        scratch_types=[
            pltpu.VMEM((C, K), jnp.int32),
            pltpu.VMEM((C, K), jnp.int32),
            [pltpu.VMEM((K, W), jnp.float32)] * 4,
            [pltpu.SemaphoreType.DMA] * 4,
            [pltpu.SemaphoreType.DMA] * 4,
            pltpu.VMEM_SHARED((N, W), jnp.float32),
        ],
    )
    def k(h_hbm, isrc_hbm, idst_hbm, z_hbm, out_hbm,
          isrc_v, idst_v, rbufs, gsems, tsems, acc_sm):
        c = lax.axis_index("c")
        s = lax.axis_index("s")
        pltpu.sync_copy(idst_hbm.at[s], idst_v)

        def gather(j, p):
            pltpu.async_copy(h_hbm.at[isrc_v.at[j]], rbufs[p], gsems[p])

        def gather_wait(p):
            pltpu.make_async_copy(
                h_hbm.at[isrc_v.at[0]], rbufs[p], gsems[p]).wait()

        def scatter(j, p):
            pltpu.async_copy(rbufs[p], acc_sm.at[idst_v.at[j]], tsems[p],
                             add=True)

        def scatter_wait(p):
            pltpu.make_async_copy(rbufs[p], acc_sm.at[idst_v.at[0]],
                                  tsems[p]).wait()

        for q in range(NP):
            pltpu.sync_copy(isrc_hbm.at[c, q, s], isrc_v)
            pltpu.sync_copy(z_hbm.at[s], acc_sm.at[pl.ds(s * RPT, RPT)])
            gather(0, 0)
            gather(1, 1)
            plsc.subcore_barrier()
            for p in range(4):
                gather_wait(p)
                scatter(p, p)
                if p < 2:
                    gather(p + 2, p + 2)
                else:
                    scatter_wait(p - 2)
                    gather(p + 2, p - 2)

            def body(ii, carry):
                j0 = 4 * ii
                for p in range(4):
                    j = j0 + p
                    gather_wait(p)
                    scatter(j, p)
                    qn = (p + 2) % 4

                    @pl.when(j + 2 < C)
                    def _():
                        scatter_wait(qn)
                        gather(j + 2, qn)
                return carry

            lax.fori_loop(1, C // 4, body, 0)
            for p in range(4):
                scatter_wait(p)
            plsc.subcore_barrier()
            pltpu.sync_copy(acc_sm.at[pl.ds(s * RPT, RPT)], out_hbm.at[c, q, s])

    return k(hq, isrcq, idst, zeros)


def _t1_body(x_ref, deg_ref, xs_ref, dinv_ref):
    deg = deg_ref[0] + deg_ref[1]
    degt = deg[:, 0:1] + 1.0
    dinv = lax.rsqrt(degt)
    xs = x_ref[...] * dinv
    xs_ref[0] = xs[:, :64]
    xs_ref[1] = xs[:, 64:]
    dinv_ref[...] = dinv


def _t2_body(a0_ref, a1_ref, x0_ref, x1_ref, dinv_ref, b1_ref, w1_ref,
             w2_ref, out_ref):
    dinv = dinv_ref[...]
    u = jnp.concatenate(
        [a0_ref[...] + x0_ref[...], a1_ref[...] + x1_ref[...]], axis=1) * dinv
    z = jnp.dot(u, w1_ref[...], preferred_element_type=jnp.float32)
    a = jnp.maximum(z + b1_ref[...], 0.0)
    h2 = jnp.dot(a, w2_ref[...], preferred_element_type=jnp.float32) * dinv
    out_ref[0] = h2[:, :64]
    out_ref[1] = h2[:, 64:]


def _t3_body(a0_ref, a1_ref, h0_ref, h1_ref, dinv_ref, b2_ref, out_ref):
    r = jnp.concatenate(
        [a0_ref[...] + h0_ref[...], a1_ref[...] + h1_ref[...]], axis=1)
    out_ref[...] = r * dinv_ref[...] + b2_ref[...]


def kernel(x, edge_index, W1, b1, W2, b2):
    src = edge_index[0].astype(jnp.int32)
    dst = edge_index[1].astype(jnp.int32)
    offs2 = jnp.arange(2, dtype=jnp.int32).reshape(2, 1) * N
    isrc2 = (src[None, None] + offs2[:, :, None]).reshape(2, 1, NT, C, K)
    idst = dst.reshape(NT, C, K)
    idst_deg = dst.reshape(2, NT, C2, K)
    ones16 = jnp.ones((K, 16), jnp.float32)
    z16 = jnp.zeros((NT, RPT, 16), jnp.float32)
    zW = jnp.zeros((NT, RPT, W), jnp.float32)
    b1r = b1.reshape(1, 256)
    b2r = b2.reshape(1, 128)

    deg = _deg_kernel(idst_deg, ones16, z16).reshape(2, N, 16)

    xs2, dinv = pl.pallas_call(
        _t1_body,
        grid=(NB,),
        in_specs=[
            pl.BlockSpec((BM, 128), lambda i: (i, 0)),
            pl.BlockSpec((2, BM, 16), lambda i: (0, i, 0)),
        ],
        out_specs=[
            pl.BlockSpec((2, BM, 64), lambda i: (0, i, 0)),
            pl.BlockSpec((BM, 1), lambda i: (i, 0)),
        ],
        out_shape=[
            jax.ShapeDtypeStruct((2, N, 64), jnp.float32),
            jax.ShapeDtypeStruct((N, 1), jnp.float32),
        ],
    )(x, deg)
    xs = xs2.reshape(2 * N, 64)

    acc1 = _edge_scatter(1, xs, isrc2, idst, zW).reshape(2 * N, 64)

    h2s = pl.pallas_call(
        _t2_body,
        grid=(NB,),
        in_specs=[
            pl.BlockSpec((BM, 64), lambda i: (i, 0)),
            pl.BlockSpec((BM, 64), lambda i: (NB + i, 0)),
            pl.BlockSpec((BM, 64), lambda i: (i, 0)),
            pl.BlockSpec((BM, 64), lambda i: (NB + i, 0)),
            pl.BlockSpec((BM, 1), lambda i: (i, 0)),
            pl.BlockSpec((1, 256), lambda i: (0, 0)),
            pl.BlockSpec((128, 256), lambda i: (0, 0)),
            pl.BlockSpec((256, 128), lambda i: (0, 0)),
        ],
        out_specs=pl.BlockSpec((2, BM, 64), lambda i: (0, i, 0)),
        out_shape=jax.ShapeDtypeStruct((2, N, 64), jnp.float32),
    )(acc1, acc1, xs, xs, dinv, b1r, W1, W2)
    h2cat = h2s.reshape(2 * N, 64)

    acc2 = _edge_scatter(1, h2cat, isrc2, idst, zW).reshape(2 * N, 64)

    return pl.pallas_call(
        _t3_body,
        grid=(NB,),
        in_specs=[
            pl.BlockSpec((BM, 64), lambda i: (i, 0)),
            pl.BlockSpec((BM, 64), lambda i: (NB + i, 0)),
            pl.BlockSpec((BM, 64), lambda i: (i, 0)),
            pl.BlockSpec((BM, 64), lambda i: (NB + i, 0)),
            pl.BlockSpec((BM, 1), lambda i: (i, 0)),
            pl.BlockSpec((1, 128), lambda i: (0, 0)),
        ],
        out_specs=pl.BlockSpec((BM, 128), lambda i: (i, 0)),
        out_shape=jax.ShapeDtypeStruct((N, 128), jnp.float32),
    )(acc2, acc2, h2cat, h2cat, dinv, b2r)
